# baseline (speedup 1.0000x reference)
import functools

import jax
import jax.numpy as jnp
from jax import lax
from jax.experimental import pallas as pl
from jax.experimental.pallas import tpu as pltpu
from jax.experimental.pallas import tpu_sc as plsc

_NC = 2
_NS = 16
_NW = _NC * _NS

_B = 4 * 8192
_D = 768
_BPW = _B // (_NW // 2)
_C = 32
_NCHUNK = _BPW // _C
_NBUF = 4


def _emb_body(idx_hbm, table_hbm, out_hbm, idx_v, bufs, sems, wsems):
    sid = lax.axis_index("s")
    wid = sid
    base = wid * _BPW
    @pl.when(lax.axis_index("c") == 0)
    def _do():
        _emb_inner(idx_hbm, table_hbm, out_hbm, idx_v, bufs, sems, wsems, base)


def _emb_inner(idx_hbm, table_hbm, out_hbm, idx_v, bufs, sems, wsems, base):
    pltpu.sync_copy(idx_hbm.at[pl.ds(base, _BPW)], idx_v)

    gathers = [None] * _NBUF
    writes = [None] * _NBUF
    for g in range(_NCHUNK + _NBUF - 1):
        if g < _NCHUNK:
            s = g % _NBUF
            if g >= _NBUF:
                writes[s].wait()
            gathers[s] = pltpu.async_copy(
                table_hbm.at[idx_v.at[pl.ds(g * _C, _C)]], bufs[s], sems[s]
            )
        d = g - (_NBUF - 1)
        if d >= 0:
            p = d % _NBUF
            gathers[p].wait()
            writes[p] = pltpu.async_copy(
                bufs[p], out_hbm.at[pl.ds(base + d * _C, _C)], wsems[p]
            )
    for p in range(_NBUF):
        writes[p].wait()


_emb_call = pl.kernel(
    _emb_body,
    out_type=jax.ShapeDtypeStruct((_B, _D), jnp.float32),
    mesh=plsc.VectorSubcoreMesh(core_axis_name="c", subcore_axis_name="s"),
    scratch_types=[
        pltpu.VMEM((_BPW,), jnp.int32),
        [pltpu.VMEM((_C, _D), jnp.float32) for _ in range(_NBUF)],
        [pltpu.SemaphoreType.DMA for _ in range(_NBUF)],
        [pltpu.SemaphoreType.DMA for _ in range(_NBUF)],
    ],
)


@jax.jit
def kernel(pos_ids, wpe):
    batch, seq = pos_ids.shape
    flat_idx = pos_ids.reshape(-1).astype(jnp.int32)
    out = _emb_call(flat_idx, wpe)
    return out.reshape(batch, seq, wpe.shape[1])


# speedup vs baseline: 1.1247x; 1.1247x over previous
import jax
import jax.numpy as jnp
from jax import lax
from jax.experimental import pallas as pl
from jax.experimental.pallas import tpu as pltpu
from jax.experimental.pallas import tpu_sc as plsc

_NC = 2
_NS = 16
_NW = _NC * _NS

_BATCH = 4
_SEQ = 8192
_B = _BATCH * _SEQ
_D = 768
_BPW = _B // _NW
_WPB = _SEQ // _BPW
_C = 32
_NCHUNK = _BPW // _C
_NBUF = 4


def _emb_body(idx_hbm, table_hbm, out_hbm, idx_v, bufs, sems, wsems):
    wid = lax.axis_index("s") * _NC + lax.axis_index("c")
    b = wid // _WPB
    col = (wid % _WPB) * _BPW
    pltpu.sync_copy(idx_hbm.at[b].at[pl.ds(col, _BPW)], idx_v)

    gathers = [None] * _NBUF
    writes = [None] * _NBUF
    for g in range(_NCHUNK + _NBUF - 1):
        if g < _NCHUNK:
            s = g % _NBUF
            if g >= _NBUF:
                writes[s].wait()
            gathers[s] = pltpu.async_copy(
                table_hbm.at[idx_v.at[pl.ds(g * _C, _C)]], bufs[s], sems[s]
            )
        d = g - (_NBUF - 1)
        if d >= 0:
            p = d % _NBUF
            gathers[p].wait()
            writes[p] = pltpu.async_copy(
                bufs[p], out_hbm.at[b].at[pl.ds(col + d * _C, _C)], wsems[p]
            )
    for p in range(_NBUF):
        writes[p].wait()


_emb_call = pl.kernel(
    _emb_body,
    out_type=jax.ShapeDtypeStruct((_BATCH, _SEQ, _D), jnp.float32),
    mesh=plsc.VectorSubcoreMesh(core_axis_name="c", subcore_axis_name="s"),
    scratch_types=[
        pltpu.VMEM((_BPW,), jnp.int32),
        [pltpu.VMEM((_C, _D), jnp.float32) for _ in range(_NBUF)],
        [pltpu.SemaphoreType.DMA for _ in range(_NBUF)],
        [pltpu.SemaphoreType.DMA for _ in range(_NBUF)],
    ],
)


@jax.jit
def kernel(pos_ids, wpe):
    return _emb_call(pos_ids.astype(jnp.int32), wpe)


# speedup vs baseline: 1.1250x; 1.0003x over previous
import jax
import jax.numpy as jnp
from jax import lax
from jax.experimental import pallas as pl
from jax.experimental.pallas import tpu as pltpu
from jax.experimental.pallas import tpu_sc as plsc

_NC = 2
_NS = 16
_NW = _NC * _NS

_BATCH = 4
_SEQ = 8192
_B = _BATCH * _SEQ
_D = 768
_BPW = _B // _NW
_WPB = _SEQ // _BPW
_C = 32
_NCHUNK = _BPW // _C
_NBUF = 4


def _emb_body(idx_hbm, table_hbm, out_hbm, idx_v, bufs, sems, wsems):
    wid = lax.axis_index("c") * _NS + lax.axis_index("s")
    b = wid // _WPB
    col = (wid % _WPB) * _BPW
    pltpu.sync_copy(idx_hbm.at[b].at[pl.ds(col, _BPW)], idx_v)

    gathers = [None] * _NBUF
    writes = [None] * _NBUF
    for g in range(_NCHUNK + _NBUF - 1):
        if g < _NCHUNK:
            s = g % _NBUF
            if g >= _NBUF:
                writes[s].wait()
            gathers[s] = pltpu.async_copy(
                table_hbm.at[idx_v.at[pl.ds(g * _C, _C)]], bufs[s], sems[s]
            )
        d = g - (_NBUF - 1)
        if d >= 0:
            p = d % _NBUF
            gathers[p].wait()
            writes[p] = pltpu.async_copy(
                bufs[p], out_hbm.at[b].at[pl.ds(col + d * _C, _C)], wsems[p]
            )
    for p in range(_NBUF):
        writes[p].wait()


_emb_call = pl.kernel(
    _emb_body,
    out_type=jax.ShapeDtypeStruct((_BATCH, _SEQ, _D), jnp.float32),
    mesh=plsc.VectorSubcoreMesh(core_axis_name="c", subcore_axis_name="s"),
    scratch_types=[
        pltpu.VMEM((_BPW,), jnp.int32),
        [pltpu.VMEM((_C, _D), jnp.float32) for _ in range(_NBUF)],
        [pltpu.SemaphoreType.DMA for _ in range(_NBUF)],
        [pltpu.SemaphoreType.DMA for _ in range(_NBUF)],
    ],
)


@jax.jit
def kernel(pos_ids, wpe):
    return _emb_call(pos_ids.astype(jnp.int32), wpe)
